# baseline (speedup 1.0000x reference)
import jax
import jax.numpy as jnp
from jax.experimental import pallas as pl
from jax.experimental.pallas import tpu as pltpu


def _ceil_to(n, m):
    return ((n + m - 1) // m) * m


_TILE = 512
_CHUNK = 512


def _fwd_body(x_ref, t1_ref, b1_ref, t2_ref, b2_ref,
              w1_ref, fb1_ref, w2_ref, fb2_ref, w3_ref, fb3_ref, o_ref):
    n4 = t1_ref.shape[1]
    q1 = n4 // 4
    q2 = t2_ref.shape[1] // 4
    b1 = b1_ref[...]
    b2 = b2_ref[...]
    fb1 = fb1_ref[...]
    fb2 = fb2_ref[...]
    fb3 = fb3_ref[...]

    xall = x_ref[...].astype(jnp.bfloat16).reshape(_TILE, 784)
    for c in range(_TILE // _CHUNK):
        rows = pl.ds(c * _CHUNK, _CHUNK)
        xc = xall[c * _CHUNK:(c + 1) * _CHUNK]

        d = jnp.dot(xc, t1_ref[...], preferred_element_type=jnp.float32)
        m = jnp.maximum(jnp.maximum(d[:, :q1], d[:, q1:2 * q1]),
                        jnp.maximum(d[:, 2 * q1:3 * q1], d[:, 3 * q1:]))
        h = jnp.maximum(m + b1, 0.0).astype(jnp.bfloat16)

        e = jnp.dot(h, t2_ref[...], preferred_element_type=jnp.float32)
        m2 = jnp.maximum(jnp.maximum(e[:, :q2], e[:, q2:2 * q2]),
                         jnp.maximum(e[:, 2 * q2:3 * q2], e[:, 3 * q2:]))
        g = jnp.maximum(m2 + b2, 0.0).astype(jnp.bfloat16)

        z = jnp.dot(g, w1_ref[...], preferred_element_type=jnp.float32)
        z = jnp.maximum(z + fb1, 0.0).astype(jnp.bfloat16)
        z = jnp.dot(z, w2_ref[...], preferred_element_type=jnp.float32)
        z = jnp.maximum(z + fb2, 0.0).astype(jnp.bfloat16)
        o = jnp.dot(z, w3_ref[...], preferred_element_type=jnp.float32)
        o_ref[rows, :] = o + fb3


def kernel(x, t1, b1, t2, b2, w1, fb1, w2, fb2, w3, fb3):
    N = x.shape[0]

    padded = _ceil_to(N, _TILE)
    x3 = x
    if padded != N:
        x3 = jnp.pad(x3, ((0, padded - N), (0, 0), (0, 0), (0, 0)))

    ncp = fb3.shape[-1]
    const = lambda a: pl.BlockSpec(a.shape, (lambda i: (0,) * a.ndim),
                                   pipeline_mode=pl.Buffered(1))

    out = pl.pallas_call(
        _fwd_body,
        out_shape=jax.ShapeDtypeStruct((padded, ncp), jnp.float32),
        grid=(padded // _TILE,),
        in_specs=[
            pl.BlockSpec((_TILE, 1, 28, 28), lambda i: (i, 0, 0, 0)),
            const(t1), const(b1), const(t2), const(b2),
            const(w1), const(fb1), const(w2), const(fb2),
            const(w3), const(fb3),
        ],
        out_specs=pl.BlockSpec((_TILE, ncp), lambda i: (i, 0)),
        compiler_params=pltpu.CompilerParams(
            dimension_semantics=("parallel",),
            vmem_limit_bytes=56 * 1024 * 1024,
        ),
    )(x3, t1, b1, t2, b2, w1, fb1, w2, fb2, w3, fb3)
    return out[:N, :10]


# speedup vs baseline: 1.4639x; 1.4639x over previous
import jax
import jax.numpy as jnp
from jax.experimental import pallas as pl
from jax.experimental.pallas import tpu as pltpu


def _ceil_to(n, m):
    return ((n + m - 1) // m) * m


_TILE = 512
_CHUNK = 512


def _fwd_body(x_ref, t1_ref, b1_ref, t2_ref, b2_ref,
              w1_ref, fb1_ref, w2_ref, fb2_ref, w3_ref, fb3_ref, o_ref):
    n4 = t1_ref.shape[1]
    q1 = n4 // 4
    q2 = t2_ref.shape[1] // 4
    b1 = b1_ref[...]
    b2 = b2_ref[...]
    fb1 = fb1_ref[...]
    fb2 = fb2_ref[...]
    fb3 = fb3_ref[...]

    xall = x_ref[...].astype(jnp.bfloat16).reshape(_TILE, 784)
    for c in range(_TILE // _CHUNK):
        rows = pl.ds(c * _CHUNK, _CHUNK)
        xc = xall[c * _CHUNK:(c + 1) * _CHUNK]

        d = jnp.dot(xc, t1_ref[...], preferred_element_type=jnp.float32)
        m = jnp.maximum(jnp.maximum(d[:, :q1], d[:, q1:2 * q1]),
                        jnp.maximum(d[:, 2 * q1:3 * q1], d[:, 3 * q1:]))
        h = jnp.maximum(m + b1, 0.0).astype(jnp.bfloat16)

        e = jnp.dot(h, t2_ref[...], preferred_element_type=jnp.float32)
        m2 = jnp.maximum(jnp.maximum(e[:, :q2], e[:, q2:2 * q2]),
                         jnp.maximum(e[:, 2 * q2:3 * q2], e[:, 3 * q2:]))
        g = jnp.maximum(m2 + b2, 0.0).astype(jnp.bfloat16)

        z = jnp.dot(g, w1_ref[...], preferred_element_type=jnp.float32)
        z = jnp.maximum(z + fb1, 0.0).astype(jnp.bfloat16)
        z = jnp.dot(z, w2_ref[...], preferred_element_type=jnp.float32)
        z = jnp.maximum(z + fb2, 0.0).astype(jnp.bfloat16)
        o = jnp.dot(z, w3_ref[...], preferred_element_type=jnp.float32)
        o_ref[rows, :] = o + fb3


def kernel(x, t1, b1, t2, b2, w1, fb1, w2, fb2, w3, fb3):
    N = x.shape[0]

    x3 = x.reshape(N, 28, 28)

    padded = _ceil_to(N, _TILE)
    if padded != N:
        x3 = jnp.pad(x3, ((0, padded - N), (0, 0), (0, 0)))

    ncp = fb3.shape[-1]
    const = lambda a: pl.BlockSpec(a.shape, (lambda i: (0,) * a.ndim),
                                   pipeline_mode=pl.Buffered(1))

    out = pl.pallas_call(
        _fwd_body,
        out_shape=jax.ShapeDtypeStruct((padded, ncp), jnp.float32),
        grid=(padded // _TILE,),
        in_specs=[
            pl.BlockSpec((_TILE, 28, 28), lambda i: (i, 0, 0)),
            const(t1), const(b1), const(t2), const(b2),
            const(w1), const(fb1), const(w2), const(fb2),
            const(w3), const(fb3),
        ],
        out_specs=pl.BlockSpec((_TILE, ncp), lambda i: (i, 0)),
        compiler_params=pltpu.CompilerParams(
            dimension_semantics=("parallel",),
            vmem_limit_bytes=56 * 1024 * 1024,
        ),
    )(x3, t1, b1, t2, b2, w1, fb1, w2, fb2, w3, fb3)
    return out[:N, :10]
